# Initial kernel scaffold; baseline (speedup 1.0000x reference)
import jax
import jax.numpy as jnp
from jax.experimental import pallas as pl


def kernel(x, edge_index, W1, b1, W2, b2, W3, b3, gamma, beta, Ws, bs, Wa, ba):
    raise NotImplementedError("write your pallas kernel here")


# speedup vs baseline: 12.8464x; 12.8464x over previous
import functools

import jax
import jax.numpy as jnp
from jax import lax
from jax.experimental import pallas as pl
from jax.experimental.pallas import tpu as pltpu
from jax.experimental.pallas import tpu_sc as plsc

F32 = jnp.float32

NC = 2
NS = 16
NW = NC * NS

K_EDGE = 80
DW = 16
ZROWS = 125


def _make_deg(N, E):
    epw = E // NW
    nchunks = epw // K_EDGE
    rpt = N // NS
    mesh = plsc.VectorSubcoreMesh(core_axis_name="c", subcore_axis_name="s")

    @functools.partial(
        pl.kernel,
        mesh=mesh,
        out_type=jax.ShapeDtypeStruct((NC, N, DW), F32),
        compiler_params=pltpu.CompilerParams(use_tc_tiling_on_sc=False),
        scratch_types=[
            pltpu.VMEM((K_EDGE,), jnp.int32),
            pltpu.VMEM((K_EDGE, DW), F32),
            pltpu.VMEM((ZROWS, DW), F32),
            pltpu.VMEM_SHARED((N, DW), F32),
        ],
    )
    def deg_kernel(dst_hbm, out_hbm, dst_v, ones_v, zbuf, acc):
        c = lax.axis_index("c")
        s = lax.axis_index("s")
        wid = s * NC + c

        def ones_row(i, _):
            ones_v[i, :] = jnp.ones((DW,), F32)
            return 0

        def zero_row(i, _):
            zbuf[i, :] = jnp.zeros((DW,), F32)
            return 0

        lax.fori_loop(0, K_EDGE, ones_row, 0)
        lax.fori_loop(0, ZROWS, zero_row, 0)

        for k in range(rpt // ZROWS):
            pltpu.sync_copy(zbuf, acc.at[pl.ds(s * rpt + k * ZROWS, ZROWS)])
        plsc.subcore_barrier()

        base = wid * epw

        def chunk(j, _):
            off = base + j * K_EDGE
            pltpu.sync_copy(dst_hbm.at[pl.ds(off, K_EDGE)], dst_v)
            pltpu.sync_copy(ones_v, acc.at[dst_v], add=True)
            return 0

        lax.fori_loop(0, nchunks, chunk, 0)
        plsc.subcore_barrier()
        pltpu.sync_copy(
            acc.at[pl.ds(s * rpt, rpt)],
            out_hbm.at[c, pl.ds(s * rpt, rpt)],
        )

    return deg_kernel


def _make_prop(N, E, H):
    epw = E // NW
    nchunks = epw // K_EDGE
    rpt = N // NS
    mesh = plsc.VectorSubcoreMesh(core_axis_name="c", subcore_axis_name="s")

    @functools.partial(
        pl.kernel,
        mesh=mesh,
        out_type=jax.ShapeDtypeStruct((NC, N, H), F32),
        compiler_params=pltpu.CompilerParams(use_tc_tiling_on_sc=False),
        scratch_types=[
            pltpu.VMEM((K_EDGE,), jnp.int32),
            pltpu.VMEM((K_EDGE,), jnp.int32),
            pltpu.VMEM((K_EDGE, H), F32),
            pltpu.VMEM((ZROWS, H), F32),
            pltpu.VMEM_SHARED((N, H), F32),
            pltpu.SemaphoreType.DMA,
        ],
    )
    def prop_kernel(g_hbm, src_hbm, dst_hbm, out_hbm,
                    src_v, dst_v, rows_v, zbuf, acc, sem):
        c = lax.axis_index("c")
        s = lax.axis_index("s")
        wid = s * NC + c

        def zero_row(i, _):
            for j in range(H // 16):
                zbuf[i, pl.ds(j * 16, 16)] = jnp.zeros((16,), F32)
            return 0

        lax.fori_loop(0, ZROWS, zero_row, 0)
        for k in range(rpt // ZROWS):
            pltpu.sync_copy(zbuf, acc.at[pl.ds(s * rpt + k * ZROWS, ZROWS)])
        plsc.subcore_barrier()

        base = wid * epw

        def chunk(j, _):
            off = base + j * K_EDGE
            pltpu.sync_copy(src_hbm.at[pl.ds(off, K_EDGE)], src_v)
            pltpu.sync_copy(dst_hbm.at[pl.ds(off, K_EDGE)], dst_v)
            pltpu.async_copy(g_hbm.at[src_v], rows_v, sem).wait()
            pltpu.sync_copy(rows_v, acc.at[dst_v], add=True)
            return 0

        lax.fori_loop(0, nchunks, chunk, 0)
        plsc.subcore_barrier()
        pltpu.sync_copy(
            acc.at[pl.ds(s * rpt, rpt)],
            out_hbm.at[c, pl.ds(s * rpt, rpt)],
        )

    return prop_kernel


def _mm1(x, W1):
    def body(x_ref, w_ref, o_ref):
        o_ref[...] = jnp.dot(x_ref[...], w_ref[...],
                             preferred_element_type=F32)
    return pl.pallas_call(
        body,
        out_shape=jax.ShapeDtypeStruct((x.shape[0], W1.shape[1]), F32),
    )(x, W1)


def _scale_g(h, degp):
    def body(h_ref, d_ref, g_ref, dinv_ref):
        deg = d_ref[0, :, 0:1] + d_ref[1, :, 0:1] + 1.0
        dinv = lax.rsqrt(jnp.maximum(deg, 1.0))
        dinv_ref[...] = dinv
        g_ref[...] = h_ref[...] * dinv
    return pl.pallas_call(
        body,
        out_shape=(
            jax.ShapeDtypeStruct(h.shape, F32),
            jax.ShapeDtypeStruct((h.shape[0], 1), F32),
        ),
    )(h, degp)


def _mid_layer(acc, g, dinv, b, gamma, beta, W):
    def body(a_ref, g_ref, dinv_ref, b_ref, ga_ref, be_ref, w_ref, o_ref):
        dinv = dinv_ref[...]
        h = dinv * (a_ref[0] + a_ref[1] + g_ref[...]) + b_ref[...]
        mu = jnp.mean(h, axis=0, keepdims=True)
        xc = h - mu
        var = jnp.mean(xc * xc, axis=0, keepdims=True)
        z = ga_ref[...] * xc * lax.rsqrt(var + 1e-5) + be_ref[...]
        z = jnp.maximum(z, 0.0)
        o_ref[...] = jnp.dot(z, w_ref[...], preferred_element_type=F32) * dinv
    return pl.pallas_call(
        body,
        out_shape=jax.ShapeDtypeStruct(g.shape, F32),
    )(acc, g, dinv, b.reshape(1, -1), gamma.reshape(1, -1),
      beta.reshape(1, -1), W)


def _final_heads(acc, g, dinv, b3, Ws, bs, Wa, ba):
    def body(a_ref, g_ref, dinv_ref, b_ref, ws_ref, bs_ref, wa_ref, ba_ref,
             s_ref, al_ref):
        h = dinv_ref[...] * (a_ref[0] + a_ref[1] + g_ref[...]) + b_ref[...]
        s_ref[...] = jnp.dot(h, ws_ref[...],
                             preferred_element_type=F32) + bs_ref[...]
        al_ref[...] = jnp.dot(h, wa_ref[...],
                              preferred_element_type=F32) + ba_ref[...]
    n = g.shape[0]
    return pl.pallas_call(
        body,
        out_shape=(
            jax.ShapeDtypeStruct((n, Ws.shape[1]), F32),
            jax.ShapeDtypeStruct((n, Wa.shape[1]), F32),
        ),
    )(acc, g, dinv, b3.reshape(1, -1), Ws, bs.reshape(1, -1),
      Wa, ba.reshape(1, -1))


def kernel(x, edge_index, W1, b1, W2, b2, W3, b3, gamma, beta, Ws, bs, Wa, ba):
    N, _ = x.shape
    E = edge_index.shape[1]
    H = W1.shape[1]
    assert E % (NW * K_EDGE) == 0 and N % (NS * ZROWS) == 0

    src = edge_index[0]
    dst = edge_index[1]

    deg_k = _make_deg(N, E)
    prop_k = _make_prop(N, E, H)

    degp = deg_k(dst)
    h1 = _mm1(x, W1)
    g1, dinv = _scale_g(h1, degp)

    acc1 = prop_k(g1, src, dst)
    g2 = _mid_layer(acc1, g1, dinv, b1, gamma, beta, W2)
    acc2 = prop_k(g2, src, dst)
    g3 = _mid_layer(acc2, g2, dinv, b2, gamma, beta, W3)
    acc3 = prop_k(g3, src, dst)
    return _final_heads(acc3, g3, dinv, b3, Ws, bs, Wa, ba)
